# Initial kernel scaffold; baseline (speedup 1.0000x reference)
import jax
import jax.numpy as jnp
from jax.experimental import pallas as pl


def kernel(x, edge_index, Wl1, bl1, Wr1, Wl2, bl2, Wr2):
    raise NotImplementedError("write your pallas kernel here")


# speedup vs baseline: 6.0070x; 6.0070x over previous
import functools

import jax
import jax.numpy as jnp
from jax import lax
from jax.experimental import pallas as pl
from jax.experimental.pallas import tpu as pltpu
from jax.experimental.pallas import tpu_sc as plsc

N = 10000
E = 320000
F = 128
CLS = 16

NC = 2
NS = 16
CHUNK = 80
EPT = E // (NC * NS)
NCHUNK = EPT // CHUNK
NPAD = 10240
ROWS_PT = NPAD // NS

_mesh = plsc.VectorSubcoreMesh(core_axis_name="c", subcore_axis_name="s")
_sc_params = pltpu.CompilerParams(use_tc_tiling_on_sc=False)


@functools.partial(
    pl.kernel,
    out_type=[
        jax.ShapeDtypeStruct((NC, NPAD, F), jnp.float32),
        jax.ShapeDtypeStruct((NC, NPAD, CLS), jnp.float32),
    ],
    mesh=_mesh,
    scratch_types=[
        pltpu.VMEM_SHARED((NPAD, F), jnp.float32),
        pltpu.VMEM_SHARED((NPAD, CLS), jnp.float32),
        pltpu.VMEM((CHUNK,), jnp.int32),
        pltpu.VMEM((CHUNK,), jnp.int32),
        pltpu.VMEM((CHUNK, F), jnp.float32),
        pltpu.VMEM((CHUNK, CLS), jnp.float32),
        pltpu.VMEM((ROWS_PT // 5, F), jnp.float32),
        pltpu.VMEM((ROWS_PT, CLS), jnp.float32),
        pltpu.SemaphoreType.DMA,
    ],
    compiler_params=_sc_params,
)
def _sc_agg1(x_hbm, src_hbm, dst_hbm, sum_out, cnt_out,
             acc_sh, cnt_sh, src_v, dst_v, rows_v, ones_v, zrow_v, zcnt_v,
             sem):
    c = lax.axis_index("c")
    s = lax.axis_index("s")
    zero16 = jnp.zeros((16,), jnp.float32)
    one16 = jnp.ones((16,), jnp.float32)

    def fill_zrow(i, _):
        for j in range(F // 16):
            zrow_v[i, pl.ds(j * 16, 16)] = zero16
        return _
    lax.fori_loop(0, ROWS_PT // 5, fill_zrow, None)

    def fill_zcnt(i, _):
        zcnt_v[i] = zero16
        return _
    lax.fori_loop(0, ROWS_PT, fill_zcnt, None)

    def fill_ones(i, _):
        ones_v[i] = one16
        return _
    lax.fori_loop(0, CHUNK, fill_ones, None)

    row0 = s * ROWS_PT
    for k in range(5):
        pltpu.sync_copy(zrow_v, acc_sh.at[pl.ds(row0 + k * (ROWS_PT // 5),
                                                ROWS_PT // 5), :])
    pltpu.sync_copy(zcnt_v, cnt_sh.at[pl.ds(row0, ROWS_PT), :])
    plsc.subcore_barrier()

    ebase = c * (E // NC) + s * EPT

    def body(i, _):
        base = ebase + i * CHUNK
        pltpu.sync_copy(src_hbm.at[pl.ds(base, CHUNK)], src_v)
        pltpu.sync_copy(dst_hbm.at[pl.ds(base, CHUNK)], dst_v)
        pltpu.async_copy(x_hbm.at[src_v], rows_v, sem).wait()
        pltpu.sync_copy(rows_v, acc_sh.at[dst_v], add=True)
        pltpu.sync_copy(ones_v, cnt_sh.at[dst_v], add=True)
        return _
    lax.fori_loop(0, NCHUNK, body, None)

    plsc.subcore_barrier()
    pltpu.sync_copy(acc_sh.at[pl.ds(row0, ROWS_PT), :],
                    sum_out.at[c, pl.ds(row0, ROWS_PT), :])
    pltpu.sync_copy(cnt_sh.at[pl.ds(row0, ROWS_PT), :],
                    cnt_out.at[c, pl.ds(row0, ROWS_PT), :])


@functools.partial(
    pl.kernel,
    out_type=[jax.ShapeDtypeStruct((NC, NPAD, CLS), jnp.float32)],
    mesh=_mesh,
    scratch_types=[
        pltpu.VMEM_SHARED((NPAD, CLS), jnp.float32),
        pltpu.VMEM((CHUNK,), jnp.int32),
        pltpu.VMEM((CHUNK,), jnp.int32),
        pltpu.VMEM((CHUNK, CLS), jnp.float32),
        pltpu.VMEM((ROWS_PT, CLS), jnp.float32),
        pltpu.SemaphoreType.DMA,
    ],
    compiler_params=_sc_params,
)
def _sc_agg2(t_hbm, src_hbm, dst_hbm, sum_out,
             acc_sh, src_v, dst_v, rows_v, zcnt_v, sem):
    c = lax.axis_index("c")
    s = lax.axis_index("s")
    zero16 = jnp.zeros((16,), jnp.float32)

    def fill_z(i, _):
        zcnt_v[i] = zero16
        return _
    lax.fori_loop(0, ROWS_PT, fill_z, None)

    row0 = s * ROWS_PT
    pltpu.sync_copy(zcnt_v, acc_sh.at[pl.ds(row0, ROWS_PT), :])
    plsc.subcore_barrier()

    ebase = c * (E // NC) + s * EPT

    def body(i, _):
        base = ebase + i * CHUNK
        pltpu.sync_copy(src_hbm.at[pl.ds(base, CHUNK)], src_v)
        pltpu.sync_copy(dst_hbm.at[pl.ds(base, CHUNK)], dst_v)
        pltpu.async_copy(t_hbm.at[src_v], rows_v, sem).wait()
        pltpu.sync_copy(rows_v, acc_sh.at[dst_v], add=True)
        return _
    lax.fori_loop(0, NCHUNK, body, None)

    plsc.subcore_barrier()
    pltpu.sync_copy(acc_sh.at[pl.ds(row0, ROWS_PT), :],
                    sum_out.at[c, pl.ds(row0, ROWS_PT), :])


ROWB = 400


def _tc1_body(sum_ref, cnt_ref, x_ref, wl1t_ref, bl1_ref, wr1t_ref,
              wl2t_ref, wr2t_ref, h2p_ref, hr2_ref):
    cnt = cnt_ref[0, :, 0:1] + cnt_ref[1, :, 0:1]
    mean = (sum_ref[0] + sum_ref[1]) / jnp.maximum(cnt, 1.0)
    h = jnp.dot(mean, wl1t_ref[...], preferred_element_type=jnp.float32)
    h += bl1_ref[...]
    h += jnp.dot(x_ref[...], wr1t_ref[...], preferred_element_type=jnp.float32)
    h = jnp.maximum(h, 0.0)
    h2p_ref[...] = jnp.dot(h, wl2t_ref[...], preferred_element_type=jnp.float32)
    hr2_ref[...] = jnp.dot(h, wr2t_ref[...], preferred_element_type=jnp.float32)


def _tc1(sums, cnts, x, wl1t, bl1, wr1t, wl2t, wr2t):
    grid = N // ROWB
    return pl.pallas_call(
        _tc1_body,
        grid=(grid,),
        in_specs=[
            pl.BlockSpec((NC, ROWB, F), lambda i: (0, i, 0)),
            pl.BlockSpec((NC, ROWB, CLS), lambda i: (0, i, 0)),
            pl.BlockSpec((ROWB, F), lambda i: (i, 0)),
            pl.BlockSpec((F, F), lambda i: (0, 0)),
            pl.BlockSpec((1, F), lambda i: (0, 0)),
            pl.BlockSpec((F, F), lambda i: (0, 0)),
            pl.BlockSpec((F, CLS), lambda i: (0, 0)),
            pl.BlockSpec((F, CLS), lambda i: (0, 0)),
        ],
        out_specs=[
            pl.BlockSpec((ROWB, CLS), lambda i: (i, 0)),
            pl.BlockSpec((ROWB, CLS), lambda i: (i, 0)),
        ],
        out_shape=[
            jax.ShapeDtypeStruct((N, CLS), jnp.float32),
            jax.ShapeDtypeStruct((N, CLS), jnp.float32),
        ],
    )(sums, cnts, x, wl1t, bl1, wr1t, wl2t, wr2t)


def _tc2_body(sum2_ref, cnt_ref, hr2_ref, bl2_ref, out_ref):
    cnt = cnt_ref[0, :, 0:1] + cnt_ref[1, :, 0:1]
    z = (sum2_ref[0] + sum2_ref[1]) / jnp.maximum(cnt, 1.0)
    z += bl2_ref[...] + hr2_ref[...]
    m = jnp.max(z, axis=1, keepdims=True)
    lse = jnp.log(jnp.sum(jnp.exp(z - m), axis=1, keepdims=True)) + m
    out_ref[...] = z - lse


def _tc2(sums2, cnts, hr2, bl2):
    grid = N // ROWB
    return pl.pallas_call(
        _tc2_body,
        grid=(grid,),
        in_specs=[
            pl.BlockSpec((NC, ROWB, CLS), lambda i: (0, i, 0)),
            pl.BlockSpec((NC, ROWB, CLS), lambda i: (0, i, 0)),
            pl.BlockSpec((ROWB, CLS), lambda i: (i, 0)),
            pl.BlockSpec((1, CLS), lambda i: (0, 0)),
        ],
        out_specs=pl.BlockSpec((ROWB, CLS), lambda i: (i, 0)),
        out_shape=jax.ShapeDtypeStruct((N, CLS), jnp.float32),
    )(sums2, cnts, hr2, bl2)


def kernel(x, edge_index, Wl1, bl1, Wr1, Wl2, bl2, Wr2):
    src = edge_index[0]
    dst = edge_index[1]
    sums, cnts = _sc_agg1(x, src, dst)
    h2p, hr2 = _tc1(sums, cnts, x,
                    Wl1.T, bl1.reshape(1, F), Wr1.T, Wl2.T, Wr2.T)
    (sums2,) = _sc_agg2(h2p, src, dst)
    return _tc2(sums2, cnts, hr2, bl2.reshape(1, CLS))


# speedup vs baseline: 13.1072x; 2.1820x over previous
import functools

import jax
import jax.numpy as jnp
from jax import lax
from jax.experimental import pallas as pl
from jax.experimental.pallas import tpu as pltpu
from jax.experimental.pallas import tpu_sc as plsc

N = 10000
E = 320000
F = 128
FH = F // 2
CLS = 16

NC = 2
NS = 16
CHUNK = 80
NROWS = E // CHUNK
ROWS_PT = N // NS

NCH1 = E // (NS * CHUNK)
NBUF1, KI1, KG1, D1 = 10, 9, 5, 4
NCH2 = E // (NC * NS * CHUNK)
NBUF2, KI2, KG2, D2 = 5, 4, 2, 2

_mesh = plsc.VectorSubcoreMesh(core_axis_name="c", subcore_axis_name="s")
_sc_params = pltpu.CompilerParams(use_tc_tiling_on_sc=False)


def _edge_ring(nchunk, nbuf, ki, kg, d, tab_hbm, srcr_hbm, src_base,
               dst_all, src_r, rows, acc_sh, cnt_sh, ones_v, a, g, s, o):
    last = nchunk - 1

    def emit(i_static, ch):
        b = i_static % nbuf
        if i_static + ki <= last:
            bi = (i_static + ki) % nbuf
            pltpu.async_copy(srcr_hbm.at[src_base + (ch + ki)], src_r[bi],
                             a[bi])
        if i_static + kg <= last:
            bg = (i_static + kg) % nbuf
            pltpu.make_async_copy(srcr_hbm.at[src_base], src_r[bg],
                                  a[bg]).wait()
            pltpu.async_copy(tab_hbm.at[src_r[bg]], rows[bg], g[bg])
        pltpu.make_async_copy(tab_hbm.at[src_r[b]], rows[b], g[b]).wait()
        pltpu.sync_copy(rows[b], acc_sh.at[dst_all.at[ch]], add=True)
        if cnt_sh is not None:
            pltpu.sync_copy(ones_v, cnt_sh.at[dst_all.at[ch]], add=True)

    for i in range(ki):
        pltpu.async_copy(srcr_hbm.at[src_base + i], src_r[i % nbuf],
                         a[i % nbuf])
    for i in range(kg):
        pltpu.make_async_copy(srcr_hbm.at[src_base], src_r[i % nbuf],
                              a[i % nbuf]).wait()
        pltpu.async_copy(tab_hbm.at[src_r[i % nbuf]], rows[i % nbuf],
                         g[i % nbuf])

    for p in range(nbuf):
        emit(p, p)

    def body(j, carry):
        base = j * nbuf
        for p in range(nbuf):
            emit(nbuf + p, base + p)
        return carry
    lax.fori_loop(1, nchunk // nbuf - 1, body, 0)

    for p in range(nbuf):
        i = nchunk - nbuf + p
        emit(i, i)


def _zero_fill(buf, nrow, ncol16):
    zero16 = jnp.zeros((16,), jnp.float32)

    def fill(i, _):
        for j in range(ncol16):
            buf[i, pl.ds(j * 16, 16)] = zero16
        return _
    lax.fori_loop(0, nrow, fill, None)


def _zero_slice(zbuf, dst_sh, row0):
    nfull = ROWS_PT // CHUNK
    rem = ROWS_PT - nfull * CHUNK
    for k in range(nfull):
        pltpu.sync_copy(zbuf, dst_sh.at[pl.ds(row0 + k * CHUNK, CHUNK), :])
    pltpu.sync_copy(zbuf.at[pl.ds(0, rem), :],
                    dst_sh.at[pl.ds(row0 + nfull * CHUNK, rem), :])


@functools.partial(
    pl.kernel,
    out_type=[
        jax.ShapeDtypeStruct((NC, N, FH), jnp.float32),
        jax.ShapeDtypeStruct((NC, N, CLS), jnp.float32),
    ],
    mesh=_mesh,
    scratch_types=[
        pltpu.VMEM_SHARED((N, FH), jnp.float32),
        pltpu.VMEM_SHARED((N, CLS), jnp.float32),
        pltpu.VMEM((NCH1, CHUNK), jnp.int32),
    ]
    + [pltpu.VMEM((CHUNK,), jnp.int32) for _ in range(NBUF1)]
    + [pltpu.VMEM((CHUNK, FH), jnp.float32) for _ in range(NBUF1)]
    + [
        pltpu.VMEM((CHUNK, CLS), jnp.float32),
        pltpu.VMEM((CHUNK, CLS), jnp.float32),
    ]
    + [pltpu.SemaphoreType.DMA for _ in range(4 * NBUF1)],
    compiler_params=_sc_params,
)
def _sc_agg1(xs_hbm, srcr2_hbm, dstr_hbm, sum_out, cnt_out, *sc):
    acc_sh, cnt_sh, dst_all = sc[0], sc[1], sc[2]
    src_r = sc[3:3 + NBUF1]
    rows = sc[3 + NBUF1:3 + 2 * NBUF1]
    ones_v, zc_v = sc[3 + 2 * NBUF1], sc[4 + 2 * NBUF1]
    sems = sc[5 + 2 * NBUF1:]
    a = sems[0:NBUF1]
    g = sems[NBUF1:2 * NBUF1]
    s = sems[2 * NBUF1:3 * NBUF1]
    o = sems[3 * NBUF1:4 * NBUF1]

    c = lax.axis_index("c")
    s_idx = lax.axis_index("s")
    one16 = jnp.ones((16,), jnp.float32)

    _zero_fill(rows[0], CHUNK, FH // 16)
    _zero_fill(zc_v, CHUNK, CLS // 16)

    def fill_ones(i, _):
        ones_v[i] = one16
        return _
    lax.fori_loop(0, CHUNK, fill_ones, None)

    row0 = s_idx * ROWS_PT
    _zero_slice(rows[0], acc_sh, row0)
    _zero_slice(zc_v, cnt_sh, row0)
    plsc.subcore_barrier()

    pltpu.sync_copy(dstr_hbm.at[pl.ds(s_idx * NCH1, NCH1), :], dst_all)
    src_base = c * NROWS + s_idx * NCH1
    _edge_ring(NCH1, NBUF1, KI1, KG1, D1, xs_hbm, srcr2_hbm, src_base,
               dst_all, src_r, rows, acc_sh, cnt_sh, ones_v, a, g, s, o)

    plsc.subcore_barrier()
    pltpu.sync_copy(acc_sh.at[pl.ds(row0, ROWS_PT), :],
                    sum_out.at[c, pl.ds(row0, ROWS_PT), :])
    pltpu.sync_copy(cnt_sh.at[pl.ds(row0, ROWS_PT), :],
                    cnt_out.at[c, pl.ds(row0, ROWS_PT), :])


@functools.partial(
    pl.kernel,
    out_type=[jax.ShapeDtypeStruct((NC, N, CLS), jnp.float32)],
    mesh=_mesh,
    scratch_types=[
        pltpu.VMEM_SHARED((N, CLS), jnp.float32),
        pltpu.VMEM((NCH2, CHUNK), jnp.int32),
    ]
    + [pltpu.VMEM((CHUNK,), jnp.int32) for _ in range(NBUF2)]
    + [pltpu.VMEM((CHUNK, CLS), jnp.float32) for _ in range(NBUF2)]
    + [pltpu.SemaphoreType.DMA for _ in range(3 * NBUF2)],
    compiler_params=_sc_params,
)
def _sc_agg2(t_hbm, srcr_hbm, dstr_hbm, sum_out, *sc):
    acc_sh, dst_all = sc[0], sc[1]
    src_r = sc[2:2 + NBUF2]
    rows = sc[2 + NBUF2:2 + 2 * NBUF2]
    sems = sc[2 + 2 * NBUF2:]
    a = sems[0:NBUF2]
    g = sems[NBUF2:2 * NBUF2]
    s = sems[2 * NBUF2:3 * NBUF2]

    c = lax.axis_index("c")
    s_idx = lax.axis_index("s")

    _zero_fill(rows[0], CHUNK, CLS // 16)
    row0 = s_idx * ROWS_PT
    _zero_slice(rows[0], acc_sh, row0)
    plsc.subcore_barrier()

    w = c * NS + s_idx
    pltpu.sync_copy(dstr_hbm.at[pl.ds(w * NCH2, NCH2), :], dst_all)
    _edge_ring(NCH2, NBUF2, KI2, KG2, D2, t_hbm, srcr_hbm, w * NCH2,
               dst_all, src_r, rows, acc_sh, None, None, a, g, s, None)

    plsc.subcore_barrier()
    pltpu.sync_copy(acc_sh.at[pl.ds(row0, ROWS_PT), :],
                    sum_out.at[c, pl.ds(row0, ROWS_PT), :])


ROWB = 400


def _tc1_body(sum_ref, cnt_ref, x_ref, wl1t_ref, bl1_ref, wr1t_ref,
              wl2t_ref, wr2t_ref, h2p_ref, hr2_ref):
    cnt = cnt_ref[0, :, 0:1]
    summed = jnp.concatenate([sum_ref[0], sum_ref[1]], axis=1)
    mean = summed / jnp.maximum(cnt, 1.0)
    h = jnp.dot(mean, wl1t_ref[...], preferred_element_type=jnp.float32)
    h += bl1_ref[...]
    h += jnp.dot(x_ref[...], wr1t_ref[...], preferred_element_type=jnp.float32)
    h = jnp.maximum(h, 0.0)
    h2p_ref[...] = jnp.dot(h, wl2t_ref[...], preferred_element_type=jnp.float32)
    hr2_ref[...] = jnp.dot(h, wr2t_ref[...], preferred_element_type=jnp.float32)


def _tc1(sums, cnts, x, wl1t, bl1, wr1t, wl2t, wr2t):
    grid = N // ROWB
    return pl.pallas_call(
        _tc1_body,
        grid=(grid,),
        in_specs=[
            pl.BlockSpec((NC, ROWB, FH), lambda i: (0, i, 0)),
            pl.BlockSpec((1, ROWB, CLS), lambda i: (0, i, 0)),
            pl.BlockSpec((ROWB, F), lambda i: (i, 0)),
            pl.BlockSpec((F, F), lambda i: (0, 0)),
            pl.BlockSpec((1, F), lambda i: (0, 0)),
            pl.BlockSpec((F, F), lambda i: (0, 0)),
            pl.BlockSpec((F, CLS), lambda i: (0, 0)),
            pl.BlockSpec((F, CLS), lambda i: (0, 0)),
        ],
        out_specs=[
            pl.BlockSpec((ROWB, CLS), lambda i: (i, 0)),
            pl.BlockSpec((ROWB, CLS), lambda i: (i, 0)),
        ],
        out_shape=[
            jax.ShapeDtypeStruct((N, CLS), jnp.float32),
            jax.ShapeDtypeStruct((N, CLS), jnp.float32),
        ],
    )(sums, cnts, x, wl1t, bl1, wr1t, wl2t, wr2t)


def _tc2_body(sum2_ref, cnt_ref, hr2_ref, bl2_ref, out_ref):
    cnt = cnt_ref[0, :, 0:1]
    z = (sum2_ref[0] + sum2_ref[1]) / jnp.maximum(cnt, 1.0)
    z += bl2_ref[...] + hr2_ref[...]
    m = jnp.max(z, axis=1, keepdims=True)
    lse = jnp.log(jnp.sum(jnp.exp(z - m), axis=1, keepdims=True)) + m
    out_ref[...] = z - lse


def _tc2(sums2, cnts, hr2, bl2):
    grid = N // ROWB
    return pl.pallas_call(
        _tc2_body,
        grid=(grid,),
        in_specs=[
            pl.BlockSpec((NC, ROWB, CLS), lambda i: (0, i, 0)),
            pl.BlockSpec((1, ROWB, CLS), lambda i: (0, i, 0)),
            pl.BlockSpec((ROWB, CLS), lambda i: (i, 0)),
            pl.BlockSpec((1, CLS), lambda i: (0, 0)),
        ],
        out_specs=pl.BlockSpec((ROWB, CLS), lambda i: (i, 0)),
        out_shape=jax.ShapeDtypeStruct((N, CLS), jnp.float32),
    )(sums2, cnts, hr2, bl2)


def kernel(x, edge_index, Wl1, bl1, Wr1, Wl2, bl2, Wr2):
    srcr = edge_index[0].reshape(NROWS, CHUNK)
    dstr = edge_index[1].reshape(NROWS, CHUNK)
    srcr2 = jnp.concatenate([srcr, srcr + N], axis=0)
    xs = jnp.concatenate([x[:, :FH], x[:, FH:]], axis=0)
    sums, cnts = _sc_agg1(xs, srcr2, dstr)
    h2p, hr2 = _tc1(sums, cnts, x,
                    Wl1.T, bl1.reshape(1, F), Wr1.T, Wl2.T, Wr2.T)
    (sums2,) = _sc_agg2(h2p, srcr, dstr)
    return _tc2(sums2, cnts, hr2, bl2.reshape(1, CLS))


# speedup vs baseline: 15.2689x; 1.1649x over previous
import functools

import jax
import jax.numpy as jnp
from jax import lax
from jax.experimental import pallas as pl
from jax.experimental.pallas import tpu as pltpu
from jax.experimental.pallas import tpu_sc as plsc

N = 10000
E = 320000
F = 128
FH = F // 2
CLS = 16

NC = 2
NS = 16
CHUNK = 80
NROWS = E // CHUNK
ROWS_PT = N // NS

NCH1 = E // (NS * CHUNK)
NBUF1, KI1, KG1, D1 = 10, 9, 5, 4
NCH2 = E // (NC * NS * CHUNK)
NBUF2, KI2, KG2, D2 = 5, 4, 2, 2

_mesh = plsc.VectorSubcoreMesh(core_axis_name="c", subcore_axis_name="s")
_sc_params = pltpu.CompilerParams(use_tc_tiling_on_sc=False)


def _edge_ring(nchunk, nbuf, ki, kg, d, tab_hbm, srcr_hbm, src_base,
               dst_all, src_r, rows, acc_sh, cnt_sh, ones_v, a, g, s, o,
               coff=None, cnt_on=None):
    last = nchunk - 1

    def emit(i_static, ch):
        b = i_static % nbuf
        if i_static + ki <= last:
            bi = (i_static + ki) % nbuf
            pltpu.async_copy(srcr_hbm.at[src_base + (ch + ki)], src_r[bi],
                             a[bi])
        if i_static + kg <= last:
            bg = (i_static + kg) % nbuf
            pltpu.make_async_copy(srcr_hbm.at[src_base], src_r[bg],
                                  a[bg]).wait()
            if coff is not None:
                for k in range(CHUNK // 16):
                    sl = pl.ds(k * 16, 16)
                    src_r[bg][sl] = src_r[bg][sl] * 2 + coff
            pltpu.async_copy(tab_hbm.at[src_r[bg]], rows[bg], g[bg])
        pltpu.make_async_copy(tab_hbm.at[src_r[b]], rows[b], g[b]).wait()
        pltpu.sync_copy(rows[b], acc_sh.at[dst_all.at[ch]], add=True)
        if cnt_sh is not None:
            @pl.when(cnt_on == (i_static % 2))
            def _():
                pltpu.sync_copy(ones_v, cnt_sh.at[dst_all.at[ch]], add=True)

    for i in range(ki):
        pltpu.async_copy(srcr_hbm.at[src_base + i], src_r[i % nbuf],
                         a[i % nbuf])
    for i in range(kg):
        pltpu.make_async_copy(srcr_hbm.at[src_base], src_r[i % nbuf],
                              a[i % nbuf]).wait()
        if coff is not None:
            for k in range(CHUNK // 16):
                sl = pl.ds(k * 16, 16)
                src_r[i % nbuf][sl] = src_r[i % nbuf][sl] * 2 + coff
        pltpu.async_copy(tab_hbm.at[src_r[i % nbuf]], rows[i % nbuf],
                         g[i % nbuf])

    for p in range(nbuf):
        emit(p, p)

    def body(j, carry):
        base = j * nbuf
        for p in range(nbuf):
            emit(nbuf + p, base + p)
        return carry
    lax.fori_loop(1, nchunk // nbuf - 1, body, 0)

    for p in range(nbuf):
        i = nchunk - nbuf + p
        emit(i, i)


def _zero_fill(buf, nrow, ncol16):
    zero16 = jnp.zeros((16,), jnp.float32)

    def fill(i, _):
        for j in range(ncol16):
            buf[i, pl.ds(j * 16, 16)] = zero16
        return _
    lax.fori_loop(0, nrow, fill, None)


def _zero_slice(zbuf, dst_sh, row0):
    nfull = ROWS_PT // CHUNK
    rem = ROWS_PT - nfull * CHUNK
    for k in range(nfull):
        pltpu.sync_copy(zbuf, dst_sh.at[pl.ds(row0 + k * CHUNK, CHUNK), :])
    pltpu.sync_copy(zbuf.at[pl.ds(0, rem), :],
                    dst_sh.at[pl.ds(row0 + nfull * CHUNK, rem), :])


@functools.partial(
    pl.kernel,
    out_type=[
        jax.ShapeDtypeStruct((NC, N, FH), jnp.float32),
        jax.ShapeDtypeStruct((NC, N, CLS), jnp.float32),
    ],
    mesh=_mesh,
    scratch_types=[
        pltpu.VMEM_SHARED((N, FH), jnp.float32),
        pltpu.VMEM_SHARED((N, CLS), jnp.float32),
        pltpu.VMEM((NCH1, CHUNK), jnp.int32),
    ]
    + [pltpu.VMEM((CHUNK,), jnp.int32) for _ in range(NBUF1)]
    + [pltpu.VMEM((CHUNK, FH), jnp.float32) for _ in range(NBUF1)]
    + [
        pltpu.VMEM((CHUNK, CLS), jnp.float32),
        pltpu.VMEM((CHUNK, CLS), jnp.float32),
    ]
    + [pltpu.SemaphoreType.DMA for _ in range(4 * NBUF1)],
    compiler_params=_sc_params,
)
def _sc_agg1(xs_hbm, srcr_hbm, dstr_hbm, sum_out, cnt_out, *sc):
    acc_sh, cnt_sh, dst_all = sc[0], sc[1], sc[2]
    src_r = sc[3:3 + NBUF1]
    rows = sc[3 + NBUF1:3 + 2 * NBUF1]
    ones_v, zc_v = sc[3 + 2 * NBUF1], sc[4 + 2 * NBUF1]
    sems = sc[5 + 2 * NBUF1:]
    a = sems[0:NBUF1]
    g = sems[NBUF1:2 * NBUF1]
    s = sems[2 * NBUF1:3 * NBUF1]
    o = sems[3 * NBUF1:4 * NBUF1]

    c = lax.axis_index("c")
    s_idx = lax.axis_index("s")
    one16 = jnp.ones((16,), jnp.float32)

    _zero_fill(rows[0], CHUNK, FH // 16)
    _zero_fill(zc_v, CHUNK, CLS // 16)

    def fill_ones(i, _):
        ones_v[i] = one16
        return _
    lax.fori_loop(0, CHUNK, fill_ones, None)

    row0 = s_idx * ROWS_PT
    _zero_slice(rows[0], acc_sh, row0)
    _zero_slice(zc_v, cnt_sh, row0)
    plsc.subcore_barrier()

    pltpu.sync_copy(dstr_hbm.at[pl.ds(s_idx * NCH1, NCH1), :], dst_all)
    _edge_ring(NCH1, NBUF1, KI1, KG1, D1, xs_hbm, srcr_hbm, s_idx * NCH1,
               dst_all, src_r, rows, acc_sh, cnt_sh, ones_v, a, g, s, o,
               coff=c, cnt_on=c)

    plsc.subcore_barrier()
    pltpu.sync_copy(acc_sh.at[pl.ds(row0, ROWS_PT), :],
                    sum_out.at[c, pl.ds(row0, ROWS_PT), :])
    pltpu.sync_copy(cnt_sh.at[pl.ds(row0, ROWS_PT), :],
                    cnt_out.at[c, pl.ds(row0, ROWS_PT), :])


@functools.partial(
    pl.kernel,
    out_type=[jax.ShapeDtypeStruct((NC, N, CLS), jnp.float32)],
    mesh=_mesh,
    scratch_types=[
        pltpu.VMEM_SHARED((N, CLS), jnp.float32),
        pltpu.VMEM((NCH2, CHUNK), jnp.int32),
    ]
    + [pltpu.VMEM((CHUNK,), jnp.int32) for _ in range(NBUF2)]
    + [pltpu.VMEM((CHUNK, CLS), jnp.float32) for _ in range(NBUF2)]
    + [pltpu.SemaphoreType.DMA for _ in range(3 * NBUF2)],
    compiler_params=_sc_params,
)
def _sc_agg2(t_hbm, srcr_hbm, dstr_hbm, sum_out, *sc):
    acc_sh, dst_all = sc[0], sc[1]
    src_r = sc[2:2 + NBUF2]
    rows = sc[2 + NBUF2:2 + 2 * NBUF2]
    sems = sc[2 + 2 * NBUF2:]
    a = sems[0:NBUF2]
    g = sems[NBUF2:2 * NBUF2]
    s = sems[2 * NBUF2:3 * NBUF2]

    c = lax.axis_index("c")
    s_idx = lax.axis_index("s")

    _zero_fill(rows[0], CHUNK, CLS // 16)
    row0 = s_idx * ROWS_PT
    _zero_slice(rows[0], acc_sh, row0)
    plsc.subcore_barrier()

    w = c * NS + s_idx
    pltpu.sync_copy(dstr_hbm.at[pl.ds(w * NCH2, NCH2), :], dst_all)
    _edge_ring(NCH2, NBUF2, KI2, KG2, D2, t_hbm, srcr_hbm, w * NCH2,
               dst_all, src_r, rows, acc_sh, None, None, a, g, s, None)

    plsc.subcore_barrier()
    pltpu.sync_copy(acc_sh.at[pl.ds(row0, ROWS_PT), :],
                    sum_out.at[c, pl.ds(row0, ROWS_PT), :])


ROWB = 400


def _tc1_body(sum_ref, cnt_ref, x_ref, wl1t_ref, bl1_ref, wr1t_ref,
              wl2t_ref, wr2t_ref, h2p_ref, hr2_ref):
    cnt = cnt_ref[0, :, 0:1] + cnt_ref[1, :, 0:1]
    summed = jnp.concatenate([sum_ref[0], sum_ref[1]], axis=1)
    mean = summed / jnp.maximum(cnt, 1.0)
    h = jnp.dot(mean, wl1t_ref[...], preferred_element_type=jnp.float32)
    h += bl1_ref[...]
    h += jnp.dot(x_ref[...], wr1t_ref[...], preferred_element_type=jnp.float32)
    h = jnp.maximum(h, 0.0)
    h2p_ref[...] = jnp.dot(h, wl2t_ref[...], preferred_element_type=jnp.float32)
    hr2_ref[...] = jnp.dot(h, wr2t_ref[...], preferred_element_type=jnp.float32)


def _tc1(sums, cnts, x, wl1t, bl1, wr1t, wl2t, wr2t):
    grid = N // ROWB
    return pl.pallas_call(
        _tc1_body,
        grid=(grid,),
        in_specs=[
            pl.BlockSpec((NC, ROWB, FH), lambda i: (0, i, 0)),
            pl.BlockSpec((NC, ROWB, CLS), lambda i: (0, i, 0)),
            pl.BlockSpec((ROWB, F), lambda i: (i, 0)),
            pl.BlockSpec((F, F), lambda i: (0, 0)),
            pl.BlockSpec((1, F), lambda i: (0, 0)),
            pl.BlockSpec((F, F), lambda i: (0, 0)),
            pl.BlockSpec((F, CLS), lambda i: (0, 0)),
            pl.BlockSpec((F, CLS), lambda i: (0, 0)),
        ],
        out_specs=[
            pl.BlockSpec((ROWB, CLS), lambda i: (i, 0)),
            pl.BlockSpec((ROWB, CLS), lambda i: (i, 0)),
        ],
        out_shape=[
            jax.ShapeDtypeStruct((N, CLS), jnp.float32),
            jax.ShapeDtypeStruct((N, CLS), jnp.float32),
        ],
    )(sums, cnts, x, wl1t, bl1, wr1t, wl2t, wr2t)


def _tc2_body(sum2_ref, cnt_ref, hr2_ref, bl2_ref, out_ref):
    cnt = cnt_ref[0, :, 0:1] + cnt_ref[1, :, 0:1]
    z = (sum2_ref[0] + sum2_ref[1]) / jnp.maximum(cnt, 1.0)
    z += bl2_ref[...] + hr2_ref[...]
    m = jnp.max(z, axis=1, keepdims=True)
    lse = jnp.log(jnp.sum(jnp.exp(z - m), axis=1, keepdims=True)) + m
    out_ref[...] = z - lse


def _tc2(sums2, cnts, hr2, bl2):
    grid = N // ROWB
    return pl.pallas_call(
        _tc2_body,
        grid=(grid,),
        in_specs=[
            pl.BlockSpec((NC, ROWB, CLS), lambda i: (0, i, 0)),
            pl.BlockSpec((NC, ROWB, CLS), lambda i: (0, i, 0)),
            pl.BlockSpec((ROWB, CLS), lambda i: (i, 0)),
            pl.BlockSpec((1, CLS), lambda i: (0, 0)),
        ],
        out_specs=pl.BlockSpec((ROWB, CLS), lambda i: (i, 0)),
        out_shape=jax.ShapeDtypeStruct((N, CLS), jnp.float32),
    )(sums2, cnts, hr2, bl2)


def kernel(x, edge_index, Wl1, bl1, Wr1, Wl2, bl2, Wr2):
    srcr = edge_index[0].reshape(NROWS, CHUNK)
    dstr = edge_index[1].reshape(NROWS, CHUNK)
    xs = x.reshape(2 * N, FH)
    sums, cnts = _sc_agg1(xs, srcr, dstr)
    h2p, hr2 = _tc1(sums, cnts, x,
                    Wl1.T, bl1.reshape(1, F), Wr1.T, Wl2.T, Wr2.T)
    (sums2,) = _sc_agg2(h2p, srcr, dstr)
    return _tc2(sums2, cnts, hr2, bl2.reshape(1, CLS))
